# Initial kernel scaffold; baseline (speedup 1.0000x reference)
import jax
import jax.numpy as jnp
from jax.experimental import pallas as pl


def kernel(h, edge_index, coord, edge_attr, We1, be1, We2, be2, Wn1, bn1, Wn2, bn2, Wc1, bc1, Wc2):
    raise NotImplementedError("write your pallas kernel here")


# speedup vs baseline: 4.3754x; 4.3754x over previous
import functools

import jax
import jax.numpy as jnp
from jax import lax
from jax.experimental import pallas as pl
from jax.experimental.pallas import tpu as pltpu
from jax.experimental.pallas import tpu_sc as plsc

NC = 2
NS = 16
NW = NC * NS
L = 16


def _make_gather(N, E, D, C):
    per_w = E // NW
    iters = per_w // C
    groups = C // L
    mesh = plsc.VectorSubcoreMesh(core_axis_name="c", subcore_axis_name="s")

    @functools.partial(
        pl.kernel,
        out_type=(
            jax.ShapeDtypeStruct((E, D), jnp.float32),
            jax.ShapeDtypeStruct((E, D), jnp.float32),
            jax.ShapeDtypeStruct((E,), jnp.float32),
            jax.ShapeDtypeStruct((E,), jnp.float32),
            jax.ShapeDtypeStruct((E,), jnp.float32),
            jax.ShapeDtypeStruct((E,), jnp.float32),
        ),
        mesh=mesh,
        scratch_types=(
            pltpu.VMEM((C,), jnp.int32),
            pltpu.VMEM((C,), jnp.int32),
            pltpu.VMEM((C,), jnp.int32),
            pltpu.VMEM((C,), jnp.int32),
            pltpu.VMEM((C, D), jnp.float32),
            pltpu.VMEM((C, D), jnp.float32),
            pltpu.VMEM((C, D), jnp.float32),
            pltpu.VMEM((C, D), jnp.float32),
            pltpu.VMEM((N,), jnp.float32),
            pltpu.VMEM((N,), jnp.float32),
            pltpu.VMEM((N,), jnp.float32),
            pltpu.VMEM((C,), jnp.float32),
            pltpu.VMEM((C,), jnp.float32),
            pltpu.VMEM((C,), jnp.float32),
            pltpu.VMEM((C,), jnp.float32),
            pltpu.VMEM((C,), jnp.float32),
            pltpu.VMEM((C,), jnp.float32),
            pltpu.VMEM((C,), jnp.float32),
            pltpu.VMEM((C,), jnp.float32),
            pltpu.SemaphoreType.DMA,
            pltpu.SemaphoreType.DMA,
            pltpu.SemaphoreType.DMA,
            pltpu.SemaphoreType.DMA,
            pltpu.SemaphoreType.DMA,
            pltpu.SemaphoreType.DMA,
        ),
        compiler_params=pltpu.CompilerParams(needs_layout_passes=False),
    )
    def gk(h_hbm, cx_hbm, cy_hbm, cz_hbm, row_hbm, col_hbm,
           hr_o, hc_o, dx_o, dy_o, dz_o, rad_o,
           ira, irb, ica, icb, bra, brb, bca, bcb, cx, cy, cz,
           dxa, dxb, dya, dyb, dza, dzb, ra, rb,
           si0, si1, sg0, sg1, so0, so1):
        wid = lax.axis_index("s") * NC + lax.axis_index("c")
        base0 = wid * per_w
        idxr = (ira, irb)
        idxc = (ica, icb)
        bufr = (bra, brb)
        bufc = (bca, bcb)
        dxs = (dxa, dxb)
        dys = (dya, dyb)
        dzs = (dza, dzb)
        rs = (ra, rb)
        sem_i = (si0, si1)
        sem_g = (sg0, sg1)
        sem_o = (so0, so1)
        pltpu.sync_copy(cx_hbm, cx)
        pltpu.sync_copy(cy_hbm, cy)
        pltpu.sync_copy(cz_hbm, cz)

        def drain_outs():
            pltpu.make_async_copy(bra, hr_o.at[pl.ds(0, C)], so0).wait()
            pltpu.make_async_copy(bca, hc_o.at[pl.ds(0, C)], so0).wait()
            for b in (dxa, dya, dza, ra):
                pltpu.make_async_copy(b, dx_o.at[pl.ds(0, C)], so0).wait()

        idx_sets = ((ira, ica, si0), (irb, icb, si1))

        def start_idx(i, j):
            b = base0 + i * C
            iri, ici, sem = idx_sets[j]
            pltpu.async_copy(row_hbm.at[pl.ds(b, C)], iri, sem)
            pltpu.async_copy(col_hbm.at[pl.ds(b, C)], ici, sem)

        def work(base, iri, ici, sem):
            pltpu.make_async_copy(
                row_hbm.at[pl.ds(0, C)], iri, sem).wait()
            pltpu.make_async_copy(
                col_hbm.at[pl.ds(0, C)], ici, sem).wait()
            cp1 = pltpu.async_copy(h_hbm.at[iri], bra, sg0)
            cp2 = pltpu.async_copy(h_hbm.at[ici], bca, sg1)
            for g in range(groups):
                ir = iri[pl.ds(g * L, L)]
                ic = ici[pl.ds(g * L, L)]
                dx = (plsc.load_gather(cx, [ir])
                      - plsc.load_gather(cx, [ic]))
                dy = (plsc.load_gather(cy, [ir])
                      - plsc.load_gather(cy, [ic]))
                dz = (plsc.load_gather(cz, [ir])
                      - plsc.load_gather(cz, [ic]))
                dxa[pl.ds(g * L, L)] = dx
                dya[pl.ds(g * L, L)] = dy
                dza[pl.ds(g * L, L)] = dz
                ra[pl.ds(g * L, L)] = dx * dx + dy * dy + dz * dz
            cp1.wait()
            cp2.wait()
            pltpu.async_copy(bra, hr_o.at[pl.ds(base, C)], so0)
            pltpu.async_copy(bca, hc_o.at[pl.ds(base, C)], so0)
            pltpu.async_copy(dxa, dx_o.at[pl.ds(base, C)], so0)
            pltpu.async_copy(dya, dy_o.at[pl.ds(base, C)], so0)
            pltpu.async_copy(dza, dz_o.at[pl.ds(base, C)], so0)
            pltpu.async_copy(ra, rad_o.at[pl.ds(base, C)], so0)

        start_idx(0, 0)

        def body(i, carry):
            base = base0 + i * C
            p = i % 2

            @pl.when(jnp.logical_and(i + 1 < iters, p == 0))
            def _():
                start_idx(i + 1, 1)

            @pl.when(jnp.logical_and(i + 1 < iters, p == 1))
            def _():
                start_idx(i + 1, 0)

            @pl.when(i > 0)
            def _():
                drain_outs()

            @pl.when(p == 0)
            def _():
                work(base, ira, ica, si0)

            @pl.when(p == 1)
            def _():
                work(base, irb, icb, si1)

            return carry

        lax.fori_loop(0, iters, body, 0)
        drain_outs()

    return gk


NP = NW // 4


def _make_scatter(N, E, D, C, CT):
    per_s = E // NS
    iters = per_s // C
    per_t = E // NP
    titers = per_t // CT
    tgroups = CT // L
    RA = (N // NS) // 8 * 8
    RL = N - (NS - 1) * RA
    mesh = plsc.VectorSubcoreMesh(core_axis_name="c", subcore_axis_name="s")

    @functools.partial(
        pl.kernel,
        out_type=(
            jax.ShapeDtypeStruct((N, D), jnp.float32),
            jax.ShapeDtypeStruct((N, D), jnp.float32),
            jax.ShapeDtypeStruct((4 * NP, N), jnp.float32),
        ),
        mesh=mesh,
        scratch_types=(
            pltpu.VMEM((C,), jnp.int32),
            pltpu.VMEM((C,), jnp.int32),
            pltpu.VMEM((C, D), jnp.float32),
            pltpu.VMEM((C, D), jnp.float32),
            pltpu.VMEM((CT,), jnp.int32),
            pltpu.VMEM((CT,), jnp.int32),
            pltpu.VMEM((CT,), jnp.float32),
            pltpu.VMEM((CT,), jnp.float32),
            pltpu.VMEM((N,), jnp.float32),
            pltpu.VMEM_SHARED((N, D), jnp.float32),
            pltpu.SemaphoreType.DMA,
            pltpu.SemaphoreType.DMA,
            pltpu.SemaphoreType.DMA,
            pltpu.SemaphoreType.DMA,
            pltpu.SemaphoreType.DMA,
            pltpu.SemaphoreType.DMA,
            pltpu.SemaphoreType.DMA,
            pltpu.SemaphoreType.DMA,
        ),
        compiler_params=pltpu.CompilerParams(needs_layout_passes=False),
    )
    def sk(eflo, efhi, trx, trY, trz, row_hbm, zN128, zN,
           agglo_o, agghi_o, part_o,
           ia, ib, efa, efb, ta, tb, va, vb, acc, accf,
           sia, sib, sea, seb, sta, stb, sva, svb):
        c = lax.axis_index("c")
        s = lax.axis_index("s")
        idx_s = (ia, ib)
        ef_s = (efa, efb)
        tid_s = (ta, tb)
        tv_s = (va, vb)
        sem_i = (sia, sib)
        sem_e = (sea, seb)
        sem_t = (sta, stb)
        sem_v = (sva, svb)

        def stripe(fn):
            @pl.when(s < NS - 1)
            def _():
                fn(s * RA, RA)

            @pl.when(s == NS - 1)
            def _():
                fn((NS - 1) * RA, RL)

        stripe(lambda r0, n: pltpu.sync_copy(
            zN128.at[pl.ds(r0, n)], accf.at[pl.ds(r0, n)]))
        pltpu.sync_copy(zN, acc)
        plsc.subcore_barrier()

        base_ef = s * per_s

        def start_ef(i, j):
            b = base_ef + i * C
            pltpu.async_copy(row_hbm.at[pl.ds(b, C)], idx_s[j], sem_i[j])

            @pl.when(c == 0)
            def _():
                pltpu.async_copy(eflo.at[pl.ds(b, C)], ef_s[j], sem_e[j])

            @pl.when(c == 1)
            def _():
                pltpu.async_copy(efhi.at[pl.ds(b, C)], ef_s[j], sem_e[j])

        start_ef(0, 0)
        start_ef(1, 1)

        def body(i2, carry):
            for j in range(2):
                i = 2 * i2 + j
                pltpu.make_async_copy(
                    row_hbm.at[pl.ds(0, C)], idx_s[j], sem_i[j]).wait()
                pltpu.make_async_copy(
                    eflo.at[pl.ds(0, C)], ef_s[j], sem_e[j]).wait()
                pltpu.sync_copy(ef_s[j], accf.at[idx_s[j]], add=True)

                @pl.when(i + 2 < iters)
                def _():
                    start_ef(i + 2, j)
            return carry

        lax.fori_loop(0, iters // 2, body, 0)

        wid = c * NS + s
        k = wid % 4
        t = wid // 4
        base_tr = t * per_t

        def start_tr(i, j):
            b = base_tr + i * CT
            pltpu.async_copy(row_hbm.at[pl.ds(b, CT)], tid_s[j], sem_t[j])

            @pl.when(k == 0)
            def _():
                pltpu.async_copy(trx.at[pl.ds(b, CT)], tv_s[j], sem_v[j])

            @pl.when(k == 1)
            def _():
                pltpu.async_copy(trY.at[pl.ds(b, CT)], tv_s[j], sem_v[j])

            @pl.when(k == 2)
            def _():
                pltpu.async_copy(trz.at[pl.ds(b, CT)], tv_s[j], sem_v[j])

        start_tr(0, 0)
        start_tr(1, 1)
        ones = jnp.full((L,), 1.0, jnp.float32)

        def tbody(i2, carry):
            for j in range(2):
                i = 2 * i2 + j
                pltpu.make_async_copy(
                    row_hbm.at[pl.ds(0, CT)], tid_s[j], sem_t[j]).wait()

                @pl.when(k < 3)
                def _():
                    pltpu.make_async_copy(
                        trx.at[pl.ds(0, CT)], tv_s[j], sem_v[j]).wait()

                for g in range(tgroups):
                    iv = tid_s[j][pl.ds(g * L, L)]

                    @pl.when(k < 3)
                    def _():
                        plsc.addupdate_scatter(
                            acc, [iv], tv_s[j][pl.ds(g * L, L)])

                    @pl.when(k == 3)
                    def _():
                        plsc.addupdate_scatter(acc, [iv], ones)

                @pl.when(i + 2 < titers)
                def _():
                    start_tr(i + 2, j)
            return carry

        lax.fori_loop(0, titers // 2, tbody, 0)
        plsc.subcore_barrier()

        pltpu.sync_copy(acc, part_o.at[k * NP + t])

        @pl.when(c == 0)
        def _():
            stripe(lambda r0, n: pltpu.sync_copy(
                accf.at[pl.ds(r0, n)], agglo_o.at[pl.ds(r0, n)]))

        @pl.when(c == 1)
        def _():
            stripe(lambda r0, n: pltpu.sync_copy(
                accf.at[pl.ds(r0, n)], agghi_o.at[pl.ds(r0, n)]))

    return sk


def _edge_body(hr, hc, dx, dy, dz, rad, ea, W1a, W1b, W1e, w1r, be1,
               We2, be2, Wc1, bc1, Wc2, eflo_o, efhi_o,
               trx_o, try_o, trz_o):
    B, D = hr.shape
    f32 = jnp.float32
    bf16 = jnp.bfloat16
    rad_row = rad[...].reshape(1, B)
    z = jnp.dot(hr[...].astype(bf16), W1a[...], preferred_element_type=f32)
    z = z + jnp.dot(hc[...].astype(bf16), W1b[...], preferred_element_type=f32)
    z = z + jnp.dot(ea[...], W1e[...], preferred_element_type=f32)
    z = z + lax.dot_general(rad_row, w1r[...],
                            (((0,), (0,)), ((), ())),
                            preferred_element_type=f32)
    z = jnp.maximum(z + be1[...], 0.0).astype(bf16)
    ef = jnp.maximum(jnp.dot(z, We2[...], preferred_element_type=f32)
                     + be2[...], 0.0)
    g = jnp.maximum(jnp.dot(ef.astype(bf16), Wc1[...],
                            preferred_element_type=f32)
                    + bc1[...], 0.0)
    cs_row = lax.dot_general(Wc2[...], g,
                             (((0,), (1,)), ((), ())),
                             preferred_element_type=f32)
    cs = cs_row.reshape(B)
    trx_o[...] = jnp.clip(dx[...] * cs, -100.0, 100.0)
    try_o[...] = jnp.clip(dy[...] * cs, -100.0, 100.0)
    trz_o[...] = jnp.clip(dz[...] * cs, -100.0, 100.0)
    eflo_o[...] = ef[:, :D]
    efhi_o[...] = ef[:, D:]


def _make_edge(E, D, H, DE, B, interpret=False):
    grid = (E // B,)
    row_spec = lambda w: pl.BlockSpec((B, w), lambda i: (i, 0))
    vec_spec = pl.BlockSpec((B,), lambda i: (i,))
    full = lambda shp: pl.BlockSpec(shp, lambda i: tuple(0 for _ in shp))
    return pl.pallas_call(
        _edge_body,
        grid=grid,
        in_specs=[
            row_spec(D), row_spec(D), vec_spec, vec_spec, vec_spec,
            vec_spec, row_spec(DE),
            full((D, H)), full((D, H)), full((DE, H)), full((1, H)),
            full((1, H)), full((H, H)), full((1, H)), full((H, H)),
            full((1, H)), full((H, 1)),
        ],
        out_specs=[row_spec(D), row_spec(D), vec_spec, vec_spec, vec_spec],
        out_shape=[
            jax.ShapeDtypeStruct((E, D), jnp.float32),
            jax.ShapeDtypeStruct((E, D), jnp.float32),
            jax.ShapeDtypeStruct((E,), jnp.float32),
            jax.ShapeDtypeStruct((E,), jnp.float32),
            jax.ShapeDtypeStruct((E,), jnp.float32),
        ],
        interpret=interpret,
    )


def _node_body(h, agglo, agghi, part, cx, cy, cz, Wn1h, Wn1lo, Wn1hi,
               bn1, Wn2, bn2, ho_o, cox_o, coy_o, coz_o):
    f32 = jnp.float32
    hv = h[...]
    pre = jnp.dot(hv, Wn1h[...], preferred_element_type=f32)
    pre = pre + jnp.dot(agglo[...], Wn1lo[...], preferred_element_type=f32)
    pre = pre + jnp.dot(agghi[...], Wn1hi[...], preferred_element_type=f32)
    pre = jnp.maximum(pre + bn1[...], 0.0)
    out = jnp.dot(pre, Wn2[...], preferred_element_type=f32) + bn2[...]
    ho_o[...] = hv + out

    p = part[...]
    sx = jnp.sum(p[0 * NP:1 * NP], axis=0)
    sy = jnp.sum(p[1 * NP:2 * NP], axis=0)
    sz = jnp.sum(p[2 * NP:3 * NP], axis=0)
    cnt = jnp.maximum(jnp.sum(p[3 * NP:4 * NP], axis=0), 1.0)
    cox_o[...] = cx[...] + sx / cnt
    coy_o[...] = cy[...] + sy / cnt
    coz_o[...] = cz[...] + sz / cnt


def _make_node(N, D, H, Bn, interpret=False):
    grid = (N // Bn,)
    row_spec = lambda w: pl.BlockSpec((Bn, w), lambda i: (i, 0))
    vec_spec = pl.BlockSpec((Bn,), lambda i: (i,))
    part_spec = pl.BlockSpec((4 * NP, Bn), lambda i: (0, i))
    full = lambda shp: pl.BlockSpec(shp, lambda i: tuple(0 for _ in shp))
    return pl.pallas_call(
        _node_body,
        grid=grid,
        in_specs=[
            row_spec(D), row_spec(H // 2), row_spec(H // 2), part_spec,
            vec_spec, vec_spec, vec_spec,
            full((D, H)), full((H // 2, H)), full((H // 2, H)),
            full((1, H)), full((H, D)), full((1, D)),
        ],
        out_specs=[row_spec(D), vec_spec, vec_spec, vec_spec],
        out_shape=[
            jax.ShapeDtypeStruct((N, D), jnp.float32),
            jax.ShapeDtypeStruct((N,), jnp.float32),
            jax.ShapeDtypeStruct((N,), jnp.float32),
            jax.ShapeDtypeStruct((N,), jnp.float32),
        ],
        interpret=interpret,
    )


def kernel(h, edge_index, coord, edge_attr, We1, be1, We2, be2,
           Wn1, bn1, Wn2, bn2, Wc1, bc1, Wc2):
    N, D = h.shape
    E = edge_index.shape[1]
    H = We2.shape[0]
    DE = edge_attr.shape[1]

    row = edge_index[0]
    col = edge_index[1]
    cx = coord[:, 0]
    cy = coord[:, 1]
    cz = coord[:, 2]

    W1a = We1[:D]
    W1b = We1[D:2 * D]
    w1r = We1[2 * D:2 * D + 1]
    W1e = We1[2 * D + 1:]

    bf16 = jnp.bfloat16
    gather = _make_gather(N, E, D, C=80)
    hr, hc, dx, dy, dz, rad = gather(h, cx, cy, cz, row, col)

    edge = _make_edge(E, D, H, DE, B=512)
    eflo, efhi, trx, trY, trz = edge(
        hr, hc, dx, dy, dz, rad, edge_attr,
        W1a.astype(bf16), W1b.astype(bf16), W1e, w1r, be1.reshape(1, H),
        We2.astype(bf16), be2.reshape(1, H), Wc1.astype(bf16),
        bc1.reshape(1, H), Wc2.reshape(H, 1))

    scatter = _make_scatter(N, E, D, C=80, CT=80)
    agglo, agghi, part = scatter(
        eflo, efhi, trx, trY, trz, row,
        jnp.zeros((N, D), jnp.float32), jnp.zeros((N,), jnp.float32))

    Wn1h = Wn1[:D]
    Wn1lo = Wn1[D:D + H // 2]
    Wn1hi = Wn1[D + H // 2:]

    node = _make_node(N, D, H, Bn=N)
    h_out, cox, coy, coz = node(
        h, agglo, agghi, part, cx, cy, cz,
        Wn1h, Wn1lo, Wn1hi, bn1.reshape(1, H), Wn2, bn2.reshape(1, D))

    return (h_out, jnp.stack([cox, coy, coz], axis=1), edge_attr)
